# baseline (speedup 1.0000x reference)
import functools

import jax
import jax.numpy as jnp
from jax.experimental import pallas as pl
from jax.experimental.pallas import tpu as pltpu


def _ru(x, m):
    return ((x + m - 1) // m) * m


def _conv_tap_kernel(y_ref, w_ref, b_ref, o_ref, *, offs, cin, mo, wp, wo, act):
    y = y_ref[0]
    acc = jnp.zeros((mo, o_ref.shape[2]), jnp.float32)
    for t, off in enumerate(offs):
        acc += jnp.dot(y[off:off + mo, :], w_ref[t * cin:(t + 1) * cin, :],
                       preferred_element_type=jnp.float32)
    acc = acc + b_ref[...]
    if act == "leaky":
        acc = jnp.where(acc > 0, acc, 0.2 * acc)
    elif act == "in_leaky":
        rows = jax.lax.broadcasted_iota(jnp.int32, (mo, 1), 0)
        mask = (rows % wp) < wo
        cnt = float((mo // wp) * wo)
        mean = jnp.sum(jnp.where(mask, acc, 0.0), axis=0, keepdims=True) / cnt
        d = acc - mean
        var = jnp.sum(jnp.where(mask, d * d, 0.0), axis=0, keepdims=True) / cnt
        acc = d * jax.lax.rsqrt(var + 1e-5)
        acc = jnp.where(acc > 0, acc, 0.2 * acc)
    o_ref[0] = acc.astype(o_ref.dtype)


def _conv8_kernel(y0_ref, y1_ref, w_ref, b_ref, o_ref, *, cin2, mo, wp, wo, act):
    ys = (y0_ref[0], y1_ref[0])
    acc = jnp.zeros((mo, o_ref.shape[2]), jnp.float32)
    t = 0
    for a1 in range(2):
        for b1 in range(2):
            off = a1 * wp + b1
            for a0 in range(2):
                acc += jnp.dot(ys[a0][off:off + mo, :],
                               w_ref[t * cin2:(t + 1) * cin2, :],
                               preferred_element_type=jnp.float32)
                t += 1
    acc = acc + b_ref[...]
    if act == "in_leaky":
        rows = jax.lax.broadcasted_iota(jnp.int32, (mo, 1), 0)
        mask = (rows % wp) < wo
        cnt = float((mo // wp) * wo)
        mean = jnp.sum(jnp.where(mask, acc, 0.0), axis=0, keepdims=True) / cnt
        d = acc - mean
        var = jnp.sum(jnp.where(mask, d * d, 0.0), axis=0, keepdims=True) / cnt
        acc = d * jax.lax.rsqrt(var + 1e-5)
        acc = jnp.where(acc > 0, acc, 0.2 * acc)
    elif act == "leaky":
        acc = jnp.where(acc > 0, acc, 0.2 * acc)
    o_ref[0] = acc.astype(o_ref.dtype)


def _conv8(y0, y1, w_taps, bias, *, cin2, tn, mo, wp, wo, act, out_dtype):
    n, rp, _ = y0.shape
    co = w_taps.shape[1]
    jt = co // tn
    return pl.pallas_call(
        functools.partial(_conv8_kernel, cin2=cin2, mo=mo, wp=wp, wo=wo,
                          act=act),
        out_shape=jax.ShapeDtypeStruct((n, mo, co), out_dtype),
        grid=(n, jt),
        in_specs=[
            pl.BlockSpec((1, rp, cin2), lambda i, j: (i, 0, 0)),
            pl.BlockSpec((1, rp, cin2), lambda i, j: (i, 0, 0)),
            pl.BlockSpec((w_taps.shape[0], tn), lambda i, j: (0, j)),
            pl.BlockSpec((1, tn), lambda i, j: (0, j)),
        ],
        out_specs=pl.BlockSpec((1, mo, tn), lambda i, j: (i, 0, j)),
        compiler_params=pltpu.CompilerParams(
            dimension_semantics=("parallel", "parallel"),
            vmem_limit_bytes=100 * 1024 * 1024,
        ),
    )(y0, y1, w_taps, bias)


def _phase_split(h, s):
    n, ho, wo, c = h.shape
    p = _ru(s + 1, 8)
    xp = jnp.pad(h, ((0, 0), (1, 2 * p - 1 - ho), (1, 1), (0, 0)))
    xp = xp.astype(jnp.bfloat16)
    y0 = xp[:, 0::2].reshape(n, p * s, 2 * c)
    y1 = xp[:, 1::2].reshape(n, p * s, 2 * c)
    return y0, y1


def _conv(y_flat, w_taps, bias, *, offs, cin, tn, mo, wp, wo, act, out_dtype):
    n, rp, _ = y_flat.shape
    co = w_taps.shape[1]
    jt = co // tn
    return pl.pallas_call(
        functools.partial(_conv_tap_kernel, offs=offs, cin=cin, mo=mo,
                          wp=wp, wo=wo, act=act),
        out_shape=jax.ShapeDtypeStruct((n, mo, co), out_dtype),
        grid=(n, jt),
        in_specs=[
            pl.BlockSpec((1, rp, cin), lambda i, j: (i, 0, 0)),
            pl.BlockSpec((w_taps.shape[0], tn), lambda i, j: (0, j)),
            pl.BlockSpec((1, tn), lambda i, j: (0, j)),
        ],
        out_specs=pl.BlockSpec((1, mo, tn), lambda i, j: (i, 0, j)),
        compiler_params=pltpu.CompilerParams(
            dimension_semantics=("parallel", "parallel"),
            vmem_limit_bytes=100 * 1024 * 1024,
        ),
    )(y_flat, w_taps, bias)


def _s2d(h, pad):
    (pt, pb), (plf, prt) = pad
    hp = jnp.pad(h, ((0, 0), (pt, pb), (plf, prt), (0, 0)))
    n, hh, ww, c = hp.shape
    s = hh // 2
    y = hp.reshape(n, s, 2, s, 2, c).transpose(0, 1, 3, 2, 4, 5)
    y = y.reshape(n, s * s, 4 * c)
    rp = _ru(s * s + s + 2, 8)
    y = jnp.pad(y, ((0, 0), (0, rp - s * s), (0, 0)))
    return y.astype(jnp.bfloat16), s


def _w_s2d(w):
    o, c, _, _ = w.shape
    wt = w.reshape(o, c, 2, 2, 2, 2).transpose(2, 4, 3, 5, 1, 0)
    return wt.reshape(16 * c, o).astype(jnp.bfloat16)


def _crop(h_flat, n, ho, wp, wo, co):
    return h_flat.reshape(n, ho, wp, co)[:, :, :wo, :]


def kernel(c1_w, c1_b, c2_w, c2_b, c3_w, c3_b, c4_w, c4_b, c5_w, c5_b, x):
    n = x.shape[0]
    sym = ((1, 1), (1, 1))

    h = jnp.transpose(x, (0, 2, 3, 1)).astype(jnp.float32)

    y, s = _s2d(h, sym)
    h1 = _conv(y, _w_s2d(c1_w), c1_b.reshape(1, -1).astype(jnp.float32),
               offs=(0, 1, s, s + 1), cin=12, tn=64, mo=128 * s,
               wp=s, wo=128, act="leaky", out_dtype=jnp.bfloat16)

    y0, y1 = _phase_split(_crop(h1, n, 128, 129, 128, 64), 65)
    h2 = _conv8(y0, y1, _w_s2d(c2_w), c2_b.reshape(1, -1).astype(jnp.float32),
                cin2=128, tn=128, mo=64 * 65, wp=65, wo=64,
                act="in_leaky", out_dtype=jnp.bfloat16)

    y0, y1 = _phase_split(_crop(h2, n, 64, 65, 64, 128), 33)
    h3 = _conv8(y0, y1, _w_s2d(c3_w), c3_b.reshape(1, -1).astype(jnp.float32),
                cin2=256, tn=128, mo=32 * 33, wp=33, wo=32,
                act="in_leaky", out_dtype=jnp.bfloat16)

    y0, y1 = _phase_split(_crop(h3, n, 32, 33, 32, 256), 17)
    h4 = _conv8(y0, y1, _w_s2d(c4_w), c4_b.reshape(1, -1).astype(jnp.float32),
                cin2=512, tn=128, mo=16 * 17, wp=17, wo=16,
                act="in_leaky", out_dtype=jnp.bfloat16)

    h4c = _crop(h4, n, 16, 17, 16, 512)
    hp5 = jnp.pad(h4c, ((0, 0), (2, 1), (2, 1), (0, 0)))
    y5 = hp5.reshape(n, 19 * 19, 512)
    y5 = jnp.pad(y5, ((0, 0), (0, _ru(19 * 19 + 3 * 19 + 4, 8) - 361), (0, 0)))
    w5 = jnp.transpose(c5_w, (2, 3, 1, 0)).reshape(16 * 512, 1)
    w5 = jnp.pad(w5, ((0, 0), (0, 127))).astype(jnp.bfloat16)
    b5 = jnp.pad(c5_b.reshape(1, 1).astype(jnp.float32), ((0, 0), (0, 127)))
    offs5 = tuple(kh * 19 + kw for kh in range(4) for kw in range(4))
    o5 = _conv(y5, w5, b5, offs=offs5, cin=512, tn=128, mo=16 * 19,
               wp=19, wo=16, act="none", out_dtype=jnp.float32)

    out = o5[:, :, 0].reshape(n, 16, 19)[:, :, :16]
    return out[:, None, :, :]


# speedup vs baseline: 1.2102x; 1.2102x over previous
import functools

import jax
import jax.numpy as jnp
from jax.experimental import pallas as pl
from jax.experimental.pallas import tpu as pltpu


def _ru(x, m):
    return ((x + m - 1) // m) * m


def _conv_tap_kernel(y_ref, w_ref, b_ref, o_ref, *, offs, cin, mo, wp, wo, act):
    y = y_ref[0]
    acc = jnp.zeros((mo, o_ref.shape[2]), jnp.float32)
    for t, off in enumerate(offs):
        acc += jnp.dot(y[off:off + mo, :], w_ref[t * cin:(t + 1) * cin, :],
                       preferred_element_type=jnp.float32)
    acc = acc + b_ref[...]
    if act == "leaky":
        acc = jnp.where(acc > 0, acc, 0.2 * acc)
    elif act == "in_leaky":
        rows = jax.lax.broadcasted_iota(jnp.int32, (mo, 1), 0)
        mask = (rows % wp) < wo
        cnt = float((mo // wp) * wo)
        mean = jnp.sum(jnp.where(mask, acc, 0.0), axis=0, keepdims=True) / cnt
        d = acc - mean
        var = jnp.sum(jnp.where(mask, d * d, 0.0), axis=0, keepdims=True) / cnt
        acc = d * jax.lax.rsqrt(var + 1e-5)
        acc = jnp.where(acc > 0, acc, 0.2 * acc)
    o_ref[0] = acc.astype(o_ref.dtype)


def _conv(y_flat, w_taps, bias, *, offs, cin, tn, mo, wp, wo, act, out_dtype):
    n, rp, _ = y_flat.shape
    co = w_taps.shape[1]
    jt = co // tn
    return pl.pallas_call(
        functools.partial(_conv_tap_kernel, offs=offs, cin=cin, mo=mo,
                          wp=wp, wo=wo, act=act),
        out_shape=jax.ShapeDtypeStruct((n, mo, co), out_dtype),
        grid=(n, jt),
        in_specs=[
            pl.BlockSpec((1, rp, cin), lambda i, j: (i, 0, 0)),
            pl.BlockSpec((w_taps.shape[0], tn), lambda i, j: (0, j)),
            pl.BlockSpec((1, tn), lambda i, j: (0, j)),
        ],
        out_specs=pl.BlockSpec((1, mo, tn), lambda i, j: (i, 0, j)),
        compiler_params=pltpu.CompilerParams(
            dimension_semantics=("parallel", "parallel"),
            vmem_limit_bytes=100 * 1024 * 1024,
        ),
    )(y_flat, w_taps, bias)


def _s2d(h, pad, cpad=0):
    (pt, pb), (plf, prt) = pad
    hp = jnp.pad(h, ((0, 0), (pt, pb), (plf, prt), (0, 0)))
    n, hh, ww, c = hp.shape
    s = hh // 2
    y = hp.reshape(n, s, 2, s, 2, c).transpose(0, 1, 3, 2, 4, 5)
    if cpad:
        y = jnp.pad(y, ((0, 0),) * 4 + ((0, 0), (0, cpad)))
    y = y.reshape(n, s * s, 4 * (c + cpad))
    rp = _ru(s * s + s + 2, 8)
    y = jnp.pad(y, ((0, 0), (0, rp - s * s), (0, 0)))
    return y.astype(jnp.bfloat16), s


def _w_s2d(w, cpad=0, opad=0):
    o, c, _, _ = w.shape
    wt = w.transpose(2, 3, 1, 0).reshape(2, 2, 2, 2, c, o)
    wt = wt.transpose(0, 2, 1, 3, 4, 5)
    wt = jnp.pad(wt, ((0, 0),) * 4 + ((0, cpad), (0, opad)))
    return wt.reshape(16 * (c + cpad), o + opad).astype(jnp.bfloat16)


def _crop(h_flat, n, ho, wp, wo, co):
    return h_flat.reshape(n, ho, wp, co)[:, :, :wo, :]


def kernel(c1_w, c1_b, c2_w, c2_b, c3_w, c3_b, c4_w, c4_b, c5_w, c5_b, x):
    n = x.shape[0]
    f32 = jnp.float32
    sym = ((1, 1), (1, 1))

    h = jnp.transpose(x, (0, 2, 3, 1)).astype(f32)

    y, s = _s2d(h, sym)
    b1 = jnp.pad(c1_b.reshape(1, -1).astype(f32), ((0, 0), (0, 64)))
    h1 = _conv(y, _w_s2d(c1_w, opad=64), b1,
               offs=(0, 1, s, s + 1), cin=12, tn=128, mo=128 * s,
               wp=s, wo=128, act="leaky", out_dtype=jnp.bfloat16)

    y, s = _s2d(_crop(h1, n, 128, 129, 128, 128), sym, cpad=0)
    w2 = _w_s2d(jnp.pad(c2_w, ((0, 0), (0, 64), (0, 0), (0, 0))))
    h2 = _conv(y, w2, c2_b.reshape(1, -1).astype(f32),
               offs=(0, 1, s, s + 1), cin=512, tn=128, mo=64 * s,
               wp=s, wo=64, act="in_leaky", out_dtype=jnp.bfloat16)

    y, s = _s2d(_crop(h2, n, 64, 65, 64, 128), sym)
    h3 = _conv(y, _w_s2d(c3_w), c3_b.reshape(1, -1).astype(f32),
               offs=(0, 1, s, s + 1), cin=512, tn=128, mo=32 * s,
               wp=s, wo=32, act="in_leaky", out_dtype=jnp.bfloat16)

    y, s = _s2d(_crop(h3, n, 32, 33, 32, 256), sym)
    h4 = _conv(y, _w_s2d(c4_w), c4_b.reshape(1, -1).astype(f32),
               offs=(0, 1, s, s + 1), cin=1024, tn=128, mo=16 * s,
               wp=s, wo=16, act="in_leaky", out_dtype=jnp.bfloat16)

    h4c = _crop(h4, n, 16, 17, 16, 512)
    hp5 = jnp.pad(h4c, ((0, 0), (2, 1), (2, 1), (0, 0)))
    y5 = hp5.reshape(n, 19 * 19, 512)
    y5 = jnp.pad(y5, ((0, 0), (0, _ru(19 * 19 + 3 * 19 + 4, 8) - 361), (0, 0)))
    w5 = jnp.transpose(c5_w, (2, 3, 1, 0)).reshape(16 * 512, 1)
    w5 = jnp.pad(w5, ((0, 0), (0, 127))).astype(jnp.bfloat16)
    b5 = jnp.pad(c5_b.reshape(1, 1).astype(f32), ((0, 0), (0, 127)))
    offs5 = tuple(kh * 19 + kw for kh in range(4) for kw in range(4))
    o5 = _conv(y5.astype(jnp.bfloat16), w5, b5, offs=offs5, cin=512, tn=128,
               mo=16 * 19, wp=19, wo=16, act="none", out_dtype=f32)

    out = o5[:, :, 0].reshape(n, 16, 19)[:, :, :16]
    return out[:, None, :, :]


# speedup vs baseline: 2.0565x; 1.6992x over previous
import functools

import jax
import jax.numpy as jnp
from jax.experimental import pallas as pl
from jax.experimental.pallas import tpu as pltpu


def _ru(x, m):
    return ((x + m - 1) // m) * m


def _conv_tap_kernel(y_ref, w_ref, b_ref, o_ref, *, offs, cin, mo, wp, wo, act):
    y = y_ref[0]
    acc = jnp.zeros((mo, o_ref.shape[2]), jnp.float32)
    for t, off in enumerate(offs):
        acc += jnp.dot(y[off:off + mo, :], w_ref[t * cin:(t + 1) * cin, :],
                       preferred_element_type=jnp.float32)
    acc = acc + b_ref[...]
    if act == "leaky":
        acc = jnp.where(acc > 0, acc, 0.2 * acc)
    elif act == "in_leaky":
        rows = jax.lax.broadcasted_iota(jnp.int32, (mo, 1), 0)
        mask = (rows % wp) < wo
        cnt = float((mo // wp) * wo)
        mean = jnp.sum(jnp.where(mask, acc, 0.0), axis=0, keepdims=True) / cnt
        d = acc - mean
        var = jnp.sum(jnp.where(mask, d * d, 0.0), axis=0, keepdims=True) / cnt
        acc = d * jax.lax.rsqrt(var + 1e-5)
        acc = jnp.where(acc > 0, acc, 0.2 * acc)
    o_ref[0] = acc.astype(o_ref.dtype)


def _conv(y_flat, w_taps, bias, *, offs, cin, tn, mo, wp, wo, act, out_dtype):
    n, rp, _ = y_flat.shape
    co = w_taps.shape[1]
    jt = co // tn
    return pl.pallas_call(
        functools.partial(_conv_tap_kernel, offs=offs, cin=cin, mo=mo,
                          wp=wp, wo=wo, act=act),
        out_shape=jax.ShapeDtypeStruct((n, mo, co), out_dtype),
        grid=(n, jt),
        in_specs=[
            pl.BlockSpec((1, rp, cin), lambda i, j: (i, 0, 0)),
            pl.BlockSpec((w_taps.shape[0], tn), lambda i, j: (0, j)),
            pl.BlockSpec((1, tn), lambda i, j: (0, j)),
        ],
        out_specs=pl.BlockSpec((1, mo, tn), lambda i, j: (i, 0, j)),
        compiler_params=pltpu.CompilerParams(
            dimension_semantics=("parallel", "parallel"),
            vmem_limit_bytes=100 * 1024 * 1024,
        ),
    )(y_flat, w_taps, bias)


def _s2d(h, pad):
    (pt, pb), (plf, prt) = pad
    hp = jnp.pad(h, ((0, 0), (pt, pb), (plf, prt), (0, 0)))
    n, hh, ww, c = hp.shape
    s = hh // 2
    t1 = hp.reshape(n, s, 2, 2 * s, c).transpose(0, 2, 1, 3, 4)
    t1 = jax.lax.optimization_barrier(t1)
    t2 = t1.reshape(n, 2, s, s, 2, c).transpose(0, 2, 3, 1, 4, 5)
    y = t2.reshape(n, s * s, 4 * c)
    rp = _ru(s * s + s + 2, 8)
    y = jnp.pad(y, ((0, 0), (0, rp - s * s), (0, 0)))
    return y.astype(jnp.bfloat16), s


def _w_s2d(w, cpad=0, opad=0):
    o, c, _, _ = w.shape
    wt = w.transpose(2, 3, 1, 0).reshape(2, 2, 2, 2, c, o)
    wt = wt.transpose(0, 2, 1, 3, 4, 5)
    wt = jnp.pad(wt, ((0, 0),) * 4 + ((0, cpad), (0, opad)))
    return wt.reshape(16 * (c + cpad), o + opad).astype(jnp.bfloat16)


def _crop(h_flat, n, ho, wp, wo, co):
    return h_flat.reshape(n, ho, wp, co)[:, :, :wo, :]


def kernel(c1_w, c1_b, c2_w, c2_b, c3_w, c3_b, c4_w, c4_b, c5_w, c5_b, x):
    n = x.shape[0]
    f32 = jnp.float32
    sym = ((1, 1), (1, 1))

    h = jnp.transpose(x, (0, 2, 3, 1)).astype(f32)

    y, s = _s2d(h, sym)
    h1 = _conv(y, _w_s2d(c1_w), c1_b.reshape(1, -1).astype(f32),
               offs=(0, 1, s, s + 1), cin=12, tn=64, mo=128 * s,
               wp=s, wo=128, act="leaky", out_dtype=jnp.bfloat16)

    y, s = _s2d(_crop(h1, n, 128, 129, 128, 64), sym)
    h2 = _conv(y, _w_s2d(c2_w), c2_b.reshape(1, -1).astype(f32),
               offs=(0, 1, s, s + 1), cin=256, tn=128, mo=64 * s,
               wp=s, wo=64, act="in_leaky", out_dtype=jnp.bfloat16)

    y, s = _s2d(_crop(h2, n, 64, 65, 64, 128), sym)
    h3 = _conv(y, _w_s2d(c3_w), c3_b.reshape(1, -1).astype(f32),
               offs=(0, 1, s, s + 1), cin=512, tn=128, mo=32 * s,
               wp=s, wo=32, act="in_leaky", out_dtype=jnp.bfloat16)

    y, s = _s2d(_crop(h3, n, 32, 33, 32, 256), sym)
    h4 = _conv(y, _w_s2d(c4_w), c4_b.reshape(1, -1).astype(f32),
               offs=(0, 1, s, s + 1), cin=1024, tn=128, mo=16 * s,
               wp=s, wo=16, act="in_leaky", out_dtype=jnp.bfloat16)

    h4c = _crop(h4, n, 16, 17, 16, 512)
    hp5 = jnp.pad(h4c, ((0, 0), (2, 1), (2, 1), (0, 0)))
    y5 = hp5.reshape(n, 19 * 19, 512)
    y5 = jnp.pad(y5, ((0, 0), (0, _ru(19 * 19 + 3 * 19 + 4, 8) - 361), (0, 0)))
    w5 = jnp.transpose(c5_w, (2, 3, 1, 0)).reshape(16 * 512, 1)
    w5 = jnp.pad(w5, ((0, 0), (0, 127))).astype(jnp.bfloat16)
    b5 = jnp.pad(c5_b.reshape(1, 1).astype(f32), ((0, 0), (0, 127)))
    offs5 = tuple(kh * 19 + kw for kh in range(4) for kw in range(4))
    o5 = _conv(y5.astype(jnp.bfloat16), w5, b5, offs=offs5, cin=512, tn=128,
               mo=16 * 19, wp=19, wo=16, act="none", out_dtype=f32)

    out = o5[:, :, 0].reshape(n, 16, 19)[:, :, :16]
    return out[:, None, :, :]
